# baseline (speedup 1.0000x reference)
import functools

import jax
import jax.numpy as jnp
from jax import lax
from jax.experimental import pallas as pl
from jax.experimental.pallas import tpu as pltpu
from jax.experimental.pallas import tpu_sc as plsc

B = 1024
AD = 512
NF = 64
NLB = 128
NACT = 256
HID = 1024
IN1 = NLB + AD * NF + AD
RW1 = IN1 // NF

NW = 32
SPW = B // NW

NK = 66


def _sc_counts(active_as):
    mesh = plsc.VectorSubcoreMesh(core_axis_name="c", subcore_axis_name="s")

    @functools.partial(
        pl.kernel,
        out_type=jax.ShapeDtypeStruct((B, AD), jnp.float32),
        mesh=mesh,
        compiler_params=pltpu.CompilerParams(needs_layout_passes=False,
                                             use_tc_tiling_on_sc=False),
        scratch_types=[
            pltpu.VMEM((SPW, NACT), jnp.int32),
            pltpu.VMEM((SPW, AD), jnp.float32),
        ],
    )
    def k(act_hbm, cnt_hbm, act_v, cnt_v):
        wid = lax.axis_index("s") * 2 + lax.axis_index("c")
        base = wid * SPW
        pltpu.sync_copy(act_hbm.at[pl.ds(base, SPW)], act_v)
        zeros16 = jnp.zeros((16,), jnp.float32)
        ones16 = jnp.ones((16,), jnp.float32)

        def zero_body(s, _):
            for v in range(AD // 16):
                cnt_v[s, pl.ds(v * 16, 16)] = zeros16
            return 0

        lax.fori_loop(0, SPW, zero_body, 0)

        def scat_body(s, _):
            svec = jnp.full((16,), s, jnp.int32)
            for v in range(NACT // 16):
                idx = act_v[s, pl.ds(v * 16, 16)]
                plsc.addupdate_scatter(cnt_v, [svec, idx], ones16)
            return 0

        lax.fori_loop(0, SPW, scat_body, 0)
        pltpu.sync_copy(cnt_v, cnt_hbm.at[pl.ds(base, SPW)])

    return k(active_as)


def _stats_body(c_ref, ft3, flb_ref, r1_ref, r2_ref, lb_ref, ftbuf, sems):
    q = pl.program_id(0)

    def ft_copy(qq, slot):
        return pltpu.make_async_copy(ft3.at[:, pl.ds(qq, 1), :],
                                     ftbuf.at[slot], sems.at[slot])

    @pl.when(q == 0)
    def _():
        for s in range(4):
            ft_copy(s, s).start()

    slot = lax.rem(q, 4)
    ft_copy(q, slot).wait()
    fb = ftbuf[slot].reshape(B, AD)
    cb = c_ref[...]
    t = cb * fb
    ones = jnp.ones((1, B), jnp.float32)
    dn = (((1,), (0,)), ((), ()))
    r1_ref[0] = lax.dot_general(ones, t, dn,
                                precision=lax.Precision.HIGHEST,
                                preferred_element_type=jnp.float32)
    r2_ref[0] = lax.dot_general(ones, t * fb, dn,
                                precision=lax.Precision.HIGHEST,
                                preferred_element_type=jnp.float32)

    @pl.when(q <= NF - 5)
    def _():
        ft_copy(q + 4, slot).start()

    @pl.when(q == NF - 1)
    def _():
        flb = flb_ref[...]
        lb_ref[0:1, :] = jnp.sum(flb, axis=0, keepdims=True)
        lb_ref[1:2, :] = jnp.sum(flb * flb, axis=0, keepdims=True)


def _stats_call(c2d, ft3, flb):
    return pl.pallas_call(
        _stats_body,
        grid=(NF,),
        in_specs=[
            pl.BlockSpec((B, AD), lambda q: (0, 0)),
            pl.BlockSpec(memory_space=pl.ANY),
            pl.BlockSpec((B, NLB), lambda q: (0, 0)),
        ],
        out_specs=[
            pl.BlockSpec((1, 1, AD), lambda q: (q, 0, 0)),
            pl.BlockSpec((1, 1, AD), lambda q: (q, 0, 0)),
            pl.BlockSpec((2, NLB), lambda q: (0, 0)),
        ],
        out_shape=[
            jax.ShapeDtypeStruct((NF, 1, AD), jnp.float32),
            jax.ShapeDtypeStruct((NF, 1, AD), jnp.float32),
            jax.ShapeDtypeStruct((2, NLB), jnp.float32),
        ],
        scratch_shapes=[
            pltpu.VMEM((4, B, 1, AD), jnp.float32),
            pltpu.SemaphoreType.DMA((4,)),
        ],
    )(c2d, ft3, flb)


def _mm_body(asm, bsm, ft3, m, flb, act, alb, blb, ahd, bhd, b1r, lnw, lnb,
             w4, b4r, w1r, out_ref, acc, wbuf, ftbuf, wlb, wact, sems,
             ftsems, semlb, semact):
    k = pl.program_id(0)
    dn = (((1,), (0,)), ((), ()))

    def wk_copy(kk, slot):
        return pltpu.make_async_copy(
            w1r.at[pl.ds(2, AD), pl.ds(kk, 1), :], wbuf.at[slot],
            sems.at[slot])

    def ft_copy(kk, slot):
        return pltpu.make_async_copy(ft3.at[:, pl.ds(kk, 1), :],
                                     ftbuf.at[slot], ftsems.at[slot])

    @pl.when(k == 0)
    def _():
        acc[...] = jnp.zeros((B, HID), jnp.float32)
        for s in range(4):
            wk_copy(s, s).start()
            ft_copy(s, s).start()
        pltpu.make_async_copy(w1r.at[pl.ds(0, 2), :, :], wlb, semlb).start()
        pltpu.make_async_copy(w1r.at[pl.ds(RW1 - 8, 8), :, :], wact,
                              semact).start()

    @pl.when(k <= NF - 1)
    def _():
        slot = lax.rem(k, 4)
        wk_copy(k, slot).wait()
        ft_copy(k, slot).wait()
        a = asm[k]
        b = bsm[k]
        x = m[...] * (ftbuf[slot].reshape(B, AD) * a + b)
        wv = wbuf[slot].reshape(AD, HID)
        acc[...] += lax.dot_general(x.astype(jnp.bfloat16),
                                    wv.astype(jnp.bfloat16), dn,
                                    preferred_element_type=jnp.float32)

        @pl.when(k <= NF - 5)
        def _():
            wk_copy(k + 4, slot).start()
            ft_copy(k + 4, slot).start()

    @pl.when(k == NF)
    def _():
        pltpu.make_async_copy(w1r.at[pl.ds(0, 2), :, :], wlb, semlb).wait()
        x0 = jnp.concatenate(
            [flb[:, NF:] * alb[...] + blb[...],
             flb[:, :NF] * ahd[...] + bhd[...]], axis=1)
        wv = wlb[...].reshape(NLB, HID)
        acc[...] += lax.dot_general(x0.astype(jnp.bfloat16),
                                    wv.astype(jnp.bfloat16), dn,
                                    preferred_element_type=jnp.float32)

    @pl.when(k == NK - 1)
    def _():
        pltpu.make_async_copy(w1r.at[pl.ds(RW1 - 8, 8), :, :], wact,
                              semact).wait()
        wv = wact[...].reshape(AD, HID)
        acc[...] += lax.dot_general(act[...].astype(jnp.bfloat16),
                                    wv.astype(jnp.bfloat16), dn,
                                    preferred_element_type=jnp.float32)
        h = acc[...] + b1r[...]
        h = jnp.where(h > 0, h, jnp.exp(jnp.minimum(h, 0.0)) - 1.0)
        mu = jnp.mean(h, axis=1, keepdims=True)
        hc = h - mu
        var = jnp.mean(hc * hc, axis=1, keepdims=True)
        hn = hc * lax.rsqrt(var + 1e-5) * lnw[...] + lnb[...]
        out_ref[...] = lax.dot_general(
            hn.astype(jnp.bfloat16), w4[...].astype(jnp.bfloat16), dn,
            preferred_element_type=jnp.float32) + b4r[...]


def _mm_call(alpha, beta, ft3, m, flb, act, alb, blb, ahd, bhd, b1r, lnw,
             lnb, W4, b4r, W1r):
    def full(shape):
        return pl.BlockSpec(shape, lambda k: tuple(0 for _ in shape))

    return pl.pallas_call(
        _mm_body,
        grid=(NK,),
        in_specs=[
            pl.BlockSpec(memory_space=pltpu.SMEM),
            pl.BlockSpec(memory_space=pltpu.SMEM),
            pl.BlockSpec(memory_space=pl.ANY),
            full((B, AD)),
            full((B, NLB)),
            full((B, AD)),
            full((1, NF)),
            full((1, NF)),
            full((1, NF)),
            full((1, NF)),
            full((1, HID)),
            full((1, HID)),
            full((1, HID)),
            full((HID, AD)),
            full((1, AD)),
            pl.BlockSpec(memory_space=pl.ANY),
        ],
        out_specs=pl.BlockSpec((B, AD), lambda k: (0, 0)),
        out_shape=jax.ShapeDtypeStruct((B, AD), jnp.float32),
        scratch_shapes=[
            pltpu.VMEM((B, HID), jnp.float32),
            pltpu.VMEM((4, AD, 1, HID), jnp.float32),
            pltpu.VMEM((4, B, 1, AD), jnp.float32),
            pltpu.VMEM((2, NF, HID), jnp.float32),
            pltpu.VMEM((8, NF, HID), jnp.float32),
            pltpu.SemaphoreType.DMA((4,)),
            pltpu.SemaphoreType.DMA((4,)),
            pltpu.SemaphoreType.DMA,
            pltpu.SemaphoreType.DMA,
        ],
    )(alpha, beta, ft3, m, flb, act, alb, blb, ahd, bhd, b1r, lnw, lnb,
      W4, b4r, W1r)


def kernel(feature_lb, feature_as, action, active_as, bn_as_w, bn_as_b,
           bn_lb_w, bn_lb_b, W1, b1, ln1_w, ln1_b, W4, b4):
    c2d = _sc_counts(active_as)

    ft3 = jnp.transpose(feature_as, (0, 2, 1))
    r1, r2, lbs = _stats_call(c2d, ft3, feature_lb)

    S1 = jnp.sum(r1.reshape(NF, AD), axis=1)
    S2 = jnp.sum(r2.reshape(NF, AD), axis=1)
    n_as = jnp.float32(B + B * NACT)
    mean_as = (S1 + lbs[0, :NF]) / n_as
    var_as = (S2 + lbs[1, :NF]) / n_as - mean_as * mean_as
    alpha_as = bn_as_w * lax.rsqrt(var_as + 1e-5)
    beta_as = bn_as_b - mean_as * alpha_as

    mean_lb = lbs[0, NF:] / B
    var_lb = lbs[1, NF:] / B - mean_lb * mean_lb
    alpha_lb = bn_lb_w * lax.rsqrt(var_lb + 1e-5)
    beta_lb = bn_lb_b - mean_lb * alpha_lb

    m = jnp.minimum(c2d, 1.0)
    W1r = W1.reshape(RW1, NF, HID)

    return _mm_call(alpha_as, beta_as, ft3, m, feature_lb, action,
                    alpha_lb[None, :], beta_lb[None, :], alpha_as[None, :],
                    beta_as[None, :], b1[None, :], ln1_w[None, :],
                    ln1_b[None, :], W4, b4[None, :], W1r)


# speedup vs baseline: 1.2118x; 1.2118x over previous
import functools

import jax
import jax.numpy as jnp
from jax import lax
from jax.experimental import pallas as pl
from jax.experimental.pallas import tpu as pltpu
from jax.experimental.pallas import tpu_sc as plsc

B = 1024
AD = 512
NF = 64
NLB = 128
NACT = 256
HID = 1024
IN1 = NLB + AD * NF + AD
RW1 = IN1 // NF

NW = 32
SPW = B // NW

NM = NF // 2
NK = NM + 2


def _sc_counts(active_as):
    mesh = plsc.VectorSubcoreMesh(core_axis_name="c", subcore_axis_name="s")

    @functools.partial(
        pl.kernel,
        out_type=jax.ShapeDtypeStruct((B, AD), jnp.float32),
        mesh=mesh,
        compiler_params=pltpu.CompilerParams(needs_layout_passes=False,
                                             use_tc_tiling_on_sc=False),
        scratch_types=[
            pltpu.VMEM((SPW, NACT), jnp.int32),
            pltpu.VMEM((SPW, AD), jnp.float32),
        ],
    )
    def k(act_hbm, cnt_hbm, act_v, cnt_v):
        wid = lax.axis_index("s") * 2 + lax.axis_index("c")
        base = wid * SPW
        pltpu.sync_copy(act_hbm.at[pl.ds(base, SPW)], act_v)
        zeros16 = jnp.zeros((16,), jnp.float32)
        ones16 = jnp.ones((16,), jnp.float32)

        def zero_body(s, _):
            for v in range(AD // 16):
                cnt_v[s, pl.ds(v * 16, 16)] = zeros16
            return 0

        lax.fori_loop(0, SPW, zero_body, 0)

        def scat_body(s, _):
            svec = jnp.full((16,), s, jnp.int32)
            for v in range(NACT // 16):
                idx = act_v[s, pl.ds(v * 16, 16)]
                plsc.addupdate_scatter(cnt_v, [svec, idx], ones16)
            return 0

        lax.fori_loop(0, SPW, scat_body, 0)
        pltpu.sync_copy(cnt_v, cnt_hbm.at[pl.ds(base, SPW)])

    return k(active_as)


def _stats_body(c_ref, ft3, flb_ref, r1_ref, r2_ref, lb_ref, ftbuf, sems):
    q = pl.program_id(0)

    def ft_copy(qq, slot):
        return pltpu.make_async_copy(ft3.at[:, pl.ds(qq, 1), :],
                                     ftbuf.at[slot], sems.at[slot])

    @pl.when(q == 0)
    def _():
        for s in range(4):
            ft_copy(s, s).start()

    slot = lax.rem(q, 4)
    ft_copy(q, slot).wait()
    fb = ftbuf[slot].reshape(B, AD)
    cb = c_ref[...]
    t = cb * fb
    ones = jnp.ones((1, B), jnp.float32)
    dn = (((1,), (0,)), ((), ()))
    r1_ref[0] = lax.dot_general(ones, t, dn,
                                preferred_element_type=jnp.float32)
    r2_ref[0] = lax.dot_general(ones, t * fb, dn,
                                preferred_element_type=jnp.float32)

    @pl.when(q <= NF - 5)
    def _():
        ft_copy(q + 4, slot).start()

    @pl.when(q == NF - 1)
    def _():
        flb = flb_ref[...]
        lb_ref[0:1, :] = jnp.sum(flb, axis=0, keepdims=True)
        lb_ref[1:2, :] = jnp.sum(flb * flb, axis=0, keepdims=True)


def _stats_call(c2d, ft3, flb):
    return pl.pallas_call(
        _stats_body,
        grid=(NF,),
        in_specs=[
            pl.BlockSpec((B, AD), lambda q: (0, 0)),
            pl.BlockSpec(memory_space=pl.ANY),
            pl.BlockSpec((B, NLB), lambda q: (0, 0)),
        ],
        out_specs=[
            pl.BlockSpec((1, 1, AD), lambda q: (q, 0, 0)),
            pl.BlockSpec((1, 1, AD), lambda q: (q, 0, 0)),
            pl.BlockSpec((2, NLB), lambda q: (0, 0)),
        ],
        out_shape=[
            jax.ShapeDtypeStruct((NF, 1, AD), jnp.float32),
            jax.ShapeDtypeStruct((NF, 1, AD), jnp.float32),
            jax.ShapeDtypeStruct((2, NLB), jnp.float32),
        ],
        scratch_shapes=[
            pltpu.VMEM((4, B, 1, AD), jnp.float32),
            pltpu.SemaphoreType.DMA((4,)),
        ],
    )(c2d, ft3, flb)


def _mm_body(asm, bsm, ft3, m, flb, act, alb, blb, ahd, bhd, b1r, lnw, lnb,
             w4, b4r, w1r, out_ref, acc, wbuf, ftbuf, wlb, wact, sems,
             ftsems, semlb, semact):
    k = pl.program_id(0)
    dn = (((1,), (0,)), ((), ()))

    def wk_copy(g, slot):
        return pltpu.make_async_copy(
            w1r.at[pl.ds(2, AD), pl.ds(2 * g, 2), :], wbuf.at[slot],
            sems.at[slot])

    def ft_copy(g, slot):
        return pltpu.make_async_copy(ft3.at[:, pl.ds(2 * g, 2), :],
                                     ftbuf.at[slot], ftsems.at[slot])

    @pl.when(k == 0)
    def _():
        acc[...] = jnp.zeros((B, HID), jnp.float32)
        for s in range(3):
            wk_copy(s, s).start()
            ft_copy(s, s).start()
        pltpu.make_async_copy(w1r.at[pl.ds(0, 2), :, :], wlb, semlb).start()
        pltpu.make_async_copy(w1r.at[pl.ds(RW1 - 8, 8), :, :], wact,
                              semact).start()

    @pl.when(k <= NM - 1)
    def _():
        slot = lax.rem(k, 3)
        wk_copy(k, slot).wait()
        ft_copy(k, slot).wait()

        def part(j):
            a = asm[2 * k + j]
            b = bsm[2 * k + j]
            x = m[...] * (ftbuf[slot][:, j, :] * a + b)
            wv = wbuf[slot][:, j, :]
            return lax.dot_general(x.astype(jnp.bfloat16),
                                   wv.astype(jnp.bfloat16), dn,
                                   preferred_element_type=jnp.float32)

        acc[...] += part(0) + part(1)

        @pl.when(k <= NM - 4)
        def _():
            wk_copy(k + 3, slot).start()
            ft_copy(k + 3, slot).start()

    @pl.when(k == NM)
    def _():
        pltpu.make_async_copy(w1r.at[pl.ds(0, 2), :, :], wlb, semlb).wait()
        x0 = jnp.concatenate(
            [flb[:, NF:] * alb[...] + blb[...],
             flb[:, :NF] * ahd[...] + bhd[...]], axis=1)
        wv = wlb[...].reshape(NLB, HID)
        acc[...] += lax.dot_general(x0.astype(jnp.bfloat16),
                                    wv.astype(jnp.bfloat16), dn,
                                    preferred_element_type=jnp.float32)

    @pl.when(k == NK - 1)
    def _():
        pltpu.make_async_copy(w1r.at[pl.ds(RW1 - 8, 8), :, :], wact,
                              semact).wait()
        wv = wact[...].reshape(AD, HID)
        acc[...] += lax.dot_general(act[...].astype(jnp.bfloat16),
                                    wv.astype(jnp.bfloat16), dn,
                                    preferred_element_type=jnp.float32)
        h = acc[...] + b1r[...]
        h = jnp.where(h > 0, h, jnp.exp(jnp.minimum(h, 0.0)) - 1.0)
        mu = jnp.mean(h, axis=1, keepdims=True)
        hc = h - mu
        var = jnp.mean(hc * hc, axis=1, keepdims=True)
        hn = hc * lax.rsqrt(var + 1e-5) * lnw[...] + lnb[...]
        out_ref[...] = lax.dot_general(
            hn.astype(jnp.bfloat16), w4[...].astype(jnp.bfloat16), dn,
            preferred_element_type=jnp.float32) + b4r[...]


def _mm_call(alpha, beta, ft3, m, flb, act, alb, blb, ahd, bhd, b1r, lnw,
             lnb, W4, b4r, W1r):
    def full(shape):
        return pl.BlockSpec(shape, lambda k: tuple(0 for _ in shape))

    return pl.pallas_call(
        _mm_body,
        grid=(NK,),
        in_specs=[
            pl.BlockSpec(memory_space=pltpu.SMEM),
            pl.BlockSpec(memory_space=pltpu.SMEM),
            pl.BlockSpec(memory_space=pl.ANY),
            full((B, AD)),
            full((B, NLB)),
            full((B, AD)),
            full((1, NF)),
            full((1, NF)),
            full((1, NF)),
            full((1, NF)),
            full((1, HID)),
            full((1, HID)),
            full((1, HID)),
            full((HID, AD)),
            full((1, AD)),
            pl.BlockSpec(memory_space=pl.ANY),
        ],
        out_specs=pl.BlockSpec((B, AD), lambda k: (0, 0)),
        out_shape=jax.ShapeDtypeStruct((B, AD), jnp.float32),
        scratch_shapes=[
            pltpu.VMEM((B, HID), jnp.float32),
            pltpu.VMEM((3, AD, 2, HID), jnp.float32),
            pltpu.VMEM((3, B, 2, AD), jnp.float32),
            pltpu.VMEM((2, NF, HID), jnp.float32),
            pltpu.VMEM((8, NF, HID), jnp.float32),
            pltpu.SemaphoreType.DMA((3,)),
            pltpu.SemaphoreType.DMA((3,)),
            pltpu.SemaphoreType.DMA,
            pltpu.SemaphoreType.DMA,
        ],
    )(alpha, beta, ft3, m, flb, act, alb, blb, ahd, bhd, b1r, lnw, lnb,
      W4, b4r, W1r)


def kernel(feature_lb, feature_as, action, active_as, bn_as_w, bn_as_b,
           bn_lb_w, bn_lb_b, W1, b1, ln1_w, ln1_b, W4, b4):
    c2d = _sc_counts(active_as)

    ft3 = jnp.transpose(feature_as, (0, 2, 1))
    r1, r2, lbs = _stats_call(c2d, ft3, feature_lb)

    S1 = jnp.sum(r1.reshape(NF, AD), axis=1)
    S2 = jnp.sum(r2.reshape(NF, AD), axis=1)
    n_as = jnp.float32(B + B * NACT)
    mean_as = (S1 + lbs[0, :NF]) / n_as
    var_as = (S2 + lbs[1, :NF]) / n_as - mean_as * mean_as
    alpha_as = bn_as_w * lax.rsqrt(var_as + 1e-5)
    beta_as = bn_as_b - mean_as * alpha_as

    mean_lb = lbs[0, NF:] / B
    var_lb = lbs[1, NF:] / B - mean_lb * mean_lb
    alpha_lb = bn_lb_w * lax.rsqrt(var_lb + 1e-5)
    beta_lb = bn_lb_b - mean_lb * alpha_lb

    m = jnp.minimum(c2d, 1.0)
    W1r = W1.reshape(RW1, NF, HID)

    return _mm_call(alpha_as, beta_as, ft3, m, feature_lb, action,
                    alpha_lb[None, :], beta_lb[None, :], alpha_as[None, :],
                    beta_as[None, :], b1[None, :], ln1_w[None, :],
                    ln1_b[None, :], W4, b4[None, :], W1r)


# speedup vs baseline: 1.2157x; 1.0032x over previous
import functools

import jax
import jax.numpy as jnp
from jax import lax
from jax.experimental import pallas as pl
from jax.experimental.pallas import tpu as pltpu
from jax.experimental.pallas import tpu_sc as plsc

B = 1024
AD = 512
NF = 64
NLB = 128
NACT = 256
HID = 1024
IN1 = NLB + AD * NF + AD
RW1 = IN1 // NF

NW = 32
SPW = B // NW

NM = NF // 2
NK = NM + 2


def _sc_counts(active_as):
    mesh = plsc.VectorSubcoreMesh(core_axis_name="c", subcore_axis_name="s")

    @functools.partial(
        pl.kernel,
        out_type=jax.ShapeDtypeStruct((B, AD), jnp.float32),
        mesh=mesh,
        compiler_params=pltpu.CompilerParams(needs_layout_passes=False,
                                             use_tc_tiling_on_sc=False),
        scratch_types=[
            pltpu.VMEM((SPW, NACT), jnp.int32),
            pltpu.VMEM((SPW, AD), jnp.float32),
        ],
    )
    def k(act_hbm, cnt_hbm, act_v, cnt_v):
        wid = lax.axis_index("s") * 2 + lax.axis_index("c")
        base = wid * SPW
        pltpu.sync_copy(act_hbm.at[pl.ds(base, SPW)], act_v)
        zeros16 = jnp.zeros((16,), jnp.float32)
        ones16 = jnp.ones((16,), jnp.float32)

        def zero_body(s, _):
            for v in range(AD // 16):
                cnt_v[s, pl.ds(v * 16, 16)] = zeros16
            return 0

        lax.fori_loop(0, SPW, zero_body, 0)

        def scat_body(s, _):
            svec = jnp.full((16,), s, jnp.int32)
            for v in range(NACT // 16):
                idx = act_v[s, pl.ds(v * 16, 16)]
                plsc.addupdate_scatter(cnt_v, [svec, idx], ones16)
            return 0

        lax.fori_loop(0, SPW, scat_body, 0)
        pltpu.sync_copy(cnt_v, cnt_hbm.at[pl.ds(base, SPW)])

    return k(active_as)


def _stats_body(c_ref, ft3, flb_ref, r1_ref, r2_ref, lb_ref, ftbuf, sems):
    q = pl.program_id(0)

    def ft_copy(qq, slot):
        return pltpu.make_async_copy(ft3.at[:, pl.ds(qq, 1), :],
                                     ftbuf.at[slot], sems.at[slot])

    @pl.when(q == 0)
    def _():
        for s in range(4):
            ft_copy(s, s).start()

    slot = lax.rem(q, 4)
    ft_copy(q, slot).wait()
    fb = ftbuf[slot].reshape(B, AD)
    cb = c_ref[...]
    t = cb * fb
    ones = jnp.ones((1, B), jnp.float32)
    dn = (((1,), (0,)), ((), ()))
    r1_ref[0] = lax.dot_general(ones, t, dn,
                                preferred_element_type=jnp.float32)
    r2_ref[0] = lax.dot_general(ones, t * fb, dn,
                                preferred_element_type=jnp.float32)

    @pl.when(q <= NF - 5)
    def _():
        ft_copy(q + 4, slot).start()

    @pl.when(q == NF - 1)
    def _():
        flb = flb_ref[...]
        lb_ref[0:1, :] = jnp.sum(flb, axis=0, keepdims=True)
        lb_ref[1:2, :] = jnp.sum(flb * flb, axis=0, keepdims=True)


def _stats_call(c2d, ft3, flb):
    return pl.pallas_call(
        _stats_body,
        grid=(NF,),
        in_specs=[
            pl.BlockSpec((B, AD), lambda q: (0, 0)),
            pl.BlockSpec(memory_space=pl.ANY),
            pl.BlockSpec((B, NLB), lambda q: (0, 0)),
        ],
        out_specs=[
            pl.BlockSpec((1, 1, AD), lambda q: (q, 0, 0)),
            pl.BlockSpec((1, 1, AD), lambda q: (q, 0, 0)),
            pl.BlockSpec((2, NLB), lambda q: (0, 0)),
        ],
        out_shape=[
            jax.ShapeDtypeStruct((NF, 1, AD), jnp.float32),
            jax.ShapeDtypeStruct((NF, 1, AD), jnp.float32),
            jax.ShapeDtypeStruct((2, NLB), jnp.float32),
        ],
        scratch_shapes=[
            pltpu.VMEM((4, B, 1, AD), jnp.float32),
            pltpu.SemaphoreType.DMA((4,)),
        ],
    )(c2d, ft3, flb)


def _mm_body(asm, bsm, ft3, m, flb, act, alb, blb, ahd, bhd, b1r, lnw, lnb,
             w4, b4r, w1r, out_ref, acc, wbuf, ftbuf, wlb, wact, sems,
             ftsems, semlb, semact):
    k = pl.program_id(0)
    dn = (((1,), (0,)), ((), ()))

    def wk_copy(g, slot):
        return pltpu.make_async_copy(
            w1r.at[pl.ds(2, AD), pl.ds(2 * g, 2), :], wbuf.at[slot],
            sems.at[slot])

    def ft_copy(g, slot):
        return pltpu.make_async_copy(ft3.at[:, pl.ds(2 * g, 2), :],
                                     ftbuf.at[slot], ftsems.at[slot])

    @pl.when(k == 0)
    def _():
        acc[...] = jnp.zeros((B, HID), jnp.float32)
        for s in range(3):
            wk_copy(s, s).start()
            ft_copy(s, s).start()
        pltpu.make_async_copy(w1r.at[pl.ds(0, 2), :, :], wlb, semlb).start()
        pltpu.make_async_copy(w1r.at[pl.ds(RW1 - 8, 8), :, :], wact,
                              semact).start()

    @pl.when(k <= NM - 1)
    def _():
        slot = lax.rem(k, 3)
        wk_copy(k, slot).wait()
        ft_copy(k, slot).wait()

        def part(j):
            a = asm[2 * k + j]
            b = bsm[2 * k + j]
            x = m[...] * (ftbuf[slot][:, j, :] * a + b)
            wv = wbuf[slot][:, j, :]
            return lax.dot_general(x.astype(jnp.bfloat16),
                                   wv.astype(jnp.bfloat16), dn,
                                   preferred_element_type=jnp.float32)

        acc[...] += part(0)
        acc[...] += part(1)

        @pl.when(k <= NM - 4)
        def _():
            wk_copy(k + 3, slot).start()
            ft_copy(k + 3, slot).start()

    @pl.when(k == NM)
    def _():
        pltpu.make_async_copy(w1r.at[pl.ds(0, 2), :, :], wlb, semlb).wait()
        x0 = jnp.concatenate(
            [flb[:, NF:] * alb[...] + blb[...],
             flb[:, :NF] * ahd[...] + bhd[...]], axis=1)
        wv = wlb[...].reshape(NLB, HID)
        acc[...] += lax.dot_general(x0.astype(jnp.bfloat16),
                                    wv.astype(jnp.bfloat16), dn,
                                    preferred_element_type=jnp.float32)

    @pl.when(k == NK - 1)
    def _():
        pltpu.make_async_copy(w1r.at[pl.ds(RW1 - 8, 8), :, :], wact,
                              semact).wait()
        wv = wact[...].reshape(AD, HID)
        acc[...] += lax.dot_general(act[...].astype(jnp.bfloat16),
                                    wv.astype(jnp.bfloat16), dn,
                                    preferred_element_type=jnp.float32)
        h = acc[...] + b1r[...]
        h = jnp.where(h > 0, h, jnp.exp(jnp.minimum(h, 0.0)) - 1.0)
        mu = jnp.mean(h, axis=1, keepdims=True)
        hc = h - mu
        var = jnp.mean(hc * hc, axis=1, keepdims=True)
        hn = hc * lax.rsqrt(var + 1e-5) * lnw[...] + lnb[...]
        out_ref[...] = lax.dot_general(
            hn.astype(jnp.bfloat16), w4[...].astype(jnp.bfloat16), dn,
            preferred_element_type=jnp.float32) + b4r[...]


def _mm_call(alpha, beta, ft3, m, flb, act, alb, blb, ahd, bhd, b1r, lnw,
             lnb, W4, b4r, W1r):
    def full(shape):
        return pl.BlockSpec(shape, lambda k: tuple(0 for _ in shape))

    return pl.pallas_call(
        _mm_body,
        grid=(NK,),
        in_specs=[
            pl.BlockSpec(memory_space=pltpu.SMEM),
            pl.BlockSpec(memory_space=pltpu.SMEM),
            pl.BlockSpec(memory_space=pl.ANY),
            full((B, AD)),
            full((B, NLB)),
            full((B, AD)),
            full((1, NF)),
            full((1, NF)),
            full((1, NF)),
            full((1, NF)),
            full((1, HID)),
            full((1, HID)),
            full((1, HID)),
            full((HID, AD)),
            full((1, AD)),
            pl.BlockSpec(memory_space=pl.ANY),
        ],
        out_specs=pl.BlockSpec((B, AD), lambda k: (0, 0)),
        out_shape=jax.ShapeDtypeStruct((B, AD), jnp.float32),
        scratch_shapes=[
            pltpu.VMEM((B, HID), jnp.float32),
            pltpu.VMEM((3, AD, 2, HID), jnp.float32),
            pltpu.VMEM((3, B, 2, AD), jnp.float32),
            pltpu.VMEM((2, NF, HID), jnp.float32),
            pltpu.VMEM((8, NF, HID), jnp.float32),
            pltpu.SemaphoreType.DMA((3,)),
            pltpu.SemaphoreType.DMA((3,)),
            pltpu.SemaphoreType.DMA,
            pltpu.SemaphoreType.DMA,
        ],
    )(alpha, beta, ft3, m, flb, act, alb, blb, ahd, bhd, b1r, lnw, lnb,
      W4, b4r, W1r)


def kernel(feature_lb, feature_as, action, active_as, bn_as_w, bn_as_b,
           bn_lb_w, bn_lb_b, W1, b1, ln1_w, ln1_b, W4, b4):
    c2d = _sc_counts(active_as)

    ft3 = jnp.transpose(feature_as, (0, 2, 1))
    r1, r2, lbs = _stats_call(c2d, ft3, feature_lb)

    S1 = jnp.sum(r1.reshape(NF, AD), axis=1)
    S2 = jnp.sum(r2.reshape(NF, AD), axis=1)
    n_as = jnp.float32(B + B * NACT)
    mean_as = (S1 + lbs[0, :NF]) / n_as
    var_as = (S2 + lbs[1, :NF]) / n_as - mean_as * mean_as
    alpha_as = bn_as_w * lax.rsqrt(var_as + 1e-5)
    beta_as = bn_as_b - mean_as * alpha_as

    mean_lb = lbs[0, NF:] / B
    var_lb = lbs[1, NF:] / B - mean_lb * mean_lb
    alpha_lb = bn_lb_w * lax.rsqrt(var_lb + 1e-5)
    beta_lb = bn_lb_b - mean_lb * alpha_lb

    m = jnp.minimum(c2d, 1.0)
    W1r = W1.reshape(RW1, NF, HID)

    return _mm_call(alpha_as, beta_as, ft3, m, feature_lb, action,
                    alpha_lb[None, :], beta_lb[None, :], alpha_as[None, :],
                    beta_as[None, :], b1[None, :], ln1_w[None, :],
                    ln1_b[None, :], W4, b4[None, :], W1r)


# speedup vs baseline: 1.2834x; 1.0557x over previous
import functools

import jax
import jax.numpy as jnp
from jax import lax
from jax.experimental import pallas as pl
from jax.experimental.pallas import tpu as pltpu
from jax.experimental.pallas import tpu_sc as plsc

B = 1024
AD = 512
NF = 64
NLB = 128
NACT = 256
HID = 1024
IN1 = NLB + AD * NF + AD
RW1 = IN1 // NF

NW = 32
SPW = B // NW

NM = NF
NK = NM + 2


def _sc_counts(active_as):
    mesh = plsc.VectorSubcoreMesh(core_axis_name="c", subcore_axis_name="s")

    @functools.partial(
        pl.kernel,
        out_type=jax.ShapeDtypeStruct((B, AD), jnp.float32),
        mesh=mesh,
        compiler_params=pltpu.CompilerParams(needs_layout_passes=False,
                                             use_tc_tiling_on_sc=False),
        scratch_types=[
            pltpu.VMEM((SPW, NACT), jnp.int32),
            pltpu.VMEM((SPW, AD), jnp.float32),
        ],
    )
    def k(act_hbm, cnt_hbm, act_v, cnt_v):
        wid = lax.axis_index("s") * 2 + lax.axis_index("c")
        base = wid * SPW
        pltpu.sync_copy(act_hbm.at[pl.ds(base, SPW)], act_v)
        zeros16 = jnp.zeros((16,), jnp.float32)
        ones16 = jnp.ones((16,), jnp.float32)

        def zero_body(s, _):
            for v in range(AD // 16):
                cnt_v[s, pl.ds(v * 16, 16)] = zeros16
            return 0

        lax.fori_loop(0, SPW, zero_body, 0)

        def scat_body(s, _):
            svec = jnp.full((16,), s, jnp.int32)
            for v in range(NACT // 16):
                idx = act_v[s, pl.ds(v * 16, 16)]
                plsc.addupdate_scatter(cnt_v, [svec, idx], ones16)
            return 0

        lax.fori_loop(0, SPW, scat_body, 0)
        pltpu.sync_copy(cnt_v, cnt_hbm.at[pl.ds(base, SPW)])

    return k(active_as)


def _stats_body(c_ref, ft3, flb_ref, r1_ref, r2_ref, lb_ref, ftbuf, sems):
    q = pl.program_id(0)

    def ft_copy(qq, slot):
        return pltpu.make_async_copy(ft3.at[:, pl.ds(qq, 1), :],
                                     ftbuf.at[slot], sems.at[slot])

    @pl.when(q == 0)
    def _():
        for s in range(4):
            ft_copy(s, s).start()

    slot = lax.rem(q, 4)
    ft_copy(q, slot).wait()
    fb = ftbuf[slot].reshape(B, AD)
    cb = c_ref[...]
    t = cb * fb
    ones = jnp.ones((1, B), jnp.float32)
    dn = (((1,), (0,)), ((), ()))
    r1_ref[0] = lax.dot_general(ones, t, dn,
                                preferred_element_type=jnp.float32)
    r2_ref[0] = lax.dot_general(ones, t * fb, dn,
                                preferred_element_type=jnp.float32)

    @pl.when(q <= NF - 5)
    def _():
        ft_copy(q + 4, slot).start()

    @pl.when(q == NF - 1)
    def _():
        flb = flb_ref[...]
        lb_ref[0:1, :] = jnp.sum(flb, axis=0, keepdims=True)
        lb_ref[1:2, :] = jnp.sum(flb * flb, axis=0, keepdims=True)


def _stats_call(c2d, ft3, flb):
    return pl.pallas_call(
        _stats_body,
        grid=(NF,),
        in_specs=[
            pl.BlockSpec((B, AD), lambda q: (0, 0)),
            pl.BlockSpec(memory_space=pl.ANY),
            pl.BlockSpec((B, NLB), lambda q: (0, 0)),
        ],
        out_specs=[
            pl.BlockSpec((1, 1, AD), lambda q: (q, 0, 0)),
            pl.BlockSpec((1, 1, AD), lambda q: (q, 0, 0)),
            pl.BlockSpec((2, NLB), lambda q: (0, 0)),
        ],
        out_shape=[
            jax.ShapeDtypeStruct((NF, 1, AD), jnp.float32),
            jax.ShapeDtypeStruct((NF, 1, AD), jnp.float32),
            jax.ShapeDtypeStruct((2, NLB), jnp.float32),
        ],
        scratch_shapes=[
            pltpu.VMEM((4, B, 1, AD), jnp.float32),
            pltpu.SemaphoreType.DMA((4,)),
        ],
    )(c2d, ft3, flb)


def _mm_body(asm, bsm, ft3, m, flb, act, alb, blb, ahd, bhd, b1r, lnw, lnb,
             w4, b4r, w1r, out_ref, acc, wbuf, ftbuf, wlb, wact, sems,
             ftsems, semlb, semact):
    k = pl.program_id(0)
    dn = (((1,), (0,)), ((), ()))

    def wk_copy(g, slot):
        return pltpu.make_async_copy(
            w1r.at[pl.ds(2, AD), pl.ds(g, 1), :], wbuf.at[slot],
            sems.at[slot])

    def ft_copy(g, slot):
        return pltpu.make_async_copy(ft3.at[:, pl.ds(g, 1), :],
                                     ftbuf.at[slot], ftsems.at[slot])

    @pl.when(k == 0)
    def _():
        acc[...] = jnp.zeros((B, HID), jnp.float32)
        for s in range(4):
            wk_copy(s, s).start()
            ft_copy(s, s).start()
        pltpu.make_async_copy(w1r.at[pl.ds(0, 2), :, :], wlb, semlb).start()
        pltpu.make_async_copy(w1r.at[pl.ds(RW1 - 8, 8), :, :], wact,
                              semact).start()

    @pl.when(k <= NM - 1)
    def _():
        slot = lax.rem(k, 4)
        wk_copy(k, slot).wait()
        ft_copy(k, slot).wait()
        a = asm[k]
        b = bsm[k]
        x = m[...] * (ftbuf[slot].reshape(B, AD) * a + b)
        wv = wbuf[slot].reshape(AD, HID)
        acc[...] += lax.dot_general(x.astype(jnp.bfloat16),
                                    wv.astype(jnp.bfloat16), dn,
                                    preferred_element_type=jnp.float32)

        @pl.when(k <= NM - 5)
        def _():
            wk_copy(k + 4, slot).start()
            ft_copy(k + 4, slot).start()

    @pl.when(k == NM)
    def _():
        pltpu.make_async_copy(w1r.at[pl.ds(0, 2), :, :], wlb, semlb).wait()
        x0 = jnp.concatenate(
            [flb[:, NF:] * alb[...] + blb[...],
             flb[:, :NF] * ahd[...] + bhd[...]], axis=1)
        wv = wlb[...].reshape(NLB, HID)
        acc[...] += lax.dot_general(x0.astype(jnp.bfloat16),
                                    wv.astype(jnp.bfloat16), dn,
                                    preferred_element_type=jnp.float32)

    @pl.when(k == NK - 1)
    def _():
        pltpu.make_async_copy(w1r.at[pl.ds(RW1 - 8, 8), :, :], wact,
                              semact).wait()
        wv = wact[...].reshape(AD, HID)
        acc[...] += lax.dot_general(act[...].astype(jnp.bfloat16),
                                    wv.astype(jnp.bfloat16), dn,
                                    preferred_element_type=jnp.float32)
        h = acc[...] + b1r[...]
        h = jnp.where(h > 0, h, jnp.exp(jnp.minimum(h, 0.0)) - 1.0)
        mu = jnp.mean(h, axis=1, keepdims=True)
        hc = h - mu
        var = jnp.mean(hc * hc, axis=1, keepdims=True)
        hn = hc * lax.rsqrt(var + 1e-5) * lnw[...] + lnb[...]
        out_ref[...] = lax.dot_general(
            hn.astype(jnp.bfloat16), w4[...].astype(jnp.bfloat16), dn,
            preferred_element_type=jnp.float32) + b4r[...]


def _mm_call(alpha, beta, ft3, m, flb, act, alb, blb, ahd, bhd, b1r, lnw,
             lnb, W4, b4r, W1r):
    def full(shape):
        return pl.BlockSpec(shape, lambda k: tuple(0 for _ in shape))

    return pl.pallas_call(
        _mm_body,
        grid=(NK,),
        in_specs=[
            pl.BlockSpec(memory_space=pltpu.SMEM),
            pl.BlockSpec(memory_space=pltpu.SMEM),
            pl.BlockSpec(memory_space=pl.ANY),
            full((B, AD)),
            full((B, NLB)),
            full((B, AD)),
            full((1, NF)),
            full((1, NF)),
            full((1, NF)),
            full((1, NF)),
            full((1, HID)),
            full((1, HID)),
            full((1, HID)),
            full((HID, AD)),
            full((1, AD)),
            pl.BlockSpec(memory_space=pl.ANY),
        ],
        out_specs=pl.BlockSpec((B, AD), lambda k: (0, 0)),
        out_shape=jax.ShapeDtypeStruct((B, AD), jnp.float32),
        scratch_shapes=[
            pltpu.VMEM((B, HID), jnp.float32),
            pltpu.VMEM((4, AD, 1, HID), jnp.float32),
            pltpu.VMEM((4, B, 1, AD), jnp.float32),
            pltpu.VMEM((2, NF, HID), jnp.float32),
            pltpu.VMEM((8, NF, HID), jnp.float32),
            pltpu.SemaphoreType.DMA((4,)),
            pltpu.SemaphoreType.DMA((4,)),
            pltpu.SemaphoreType.DMA,
            pltpu.SemaphoreType.DMA,
        ],
    )(alpha, beta, ft3, m, flb, act, alb, blb, ahd, bhd, b1r, lnw, lnb,
      W4, b4r, W1r)


def kernel(feature_lb, feature_as, action, active_as, bn_as_w, bn_as_b,
           bn_lb_w, bn_lb_b, W1, b1, ln1_w, ln1_b, W4, b4):
    c2d = _sc_counts(active_as)

    ft3 = jnp.transpose(feature_as, (0, 2, 1))
    r1, r2, lbs = _stats_call(c2d, ft3, feature_lb)

    S1 = jnp.sum(r1.reshape(NF, AD), axis=1)
    S2 = jnp.sum(r2.reshape(NF, AD), axis=1)
    n_as = jnp.float32(B + B * NACT)
    mean_as = (S1 + lbs[0, :NF]) / n_as
    var_as = (S2 + lbs[1, :NF]) / n_as - mean_as * mean_as
    alpha_as = bn_as_w * lax.rsqrt(var_as + 1e-5)
    beta_as = bn_as_b - mean_as * alpha_as

    mean_lb = lbs[0, NF:] / B
    var_lb = lbs[1, NF:] / B - mean_lb * mean_lb
    alpha_lb = bn_lb_w * lax.rsqrt(var_lb + 1e-5)
    beta_lb = bn_lb_b - mean_lb * alpha_lb

    m = jnp.minimum(c2d, 1.0)
    W1r = W1.reshape(RW1, NF, HID)

    return _mm_call(alpha_as, beta_as, ft3, m, feature_lb, action,
                    alpha_lb[None, :], beta_lb[None, :], alpha_as[None, :],
                    beta_as[None, :], b1[None, :], ln1_w[None, :],
                    ln1_b[None, :], W4, b4[None, :], W1r)


# speedup vs baseline: 2.0578x; 1.6034x over previous
import functools

import jax
import jax.numpy as jnp
from jax import lax
from jax.experimental import pallas as pl
from jax.experimental.pallas import tpu as pltpu
from jax.experimental.pallas import tpu_sc as plsc

B = 1024
AD = 512
NF = 64
NLB = 128
NACT = 256
HID = 1024
IN1 = NLB + AD * NF + AD
RW1 = IN1 // NF

NW = 32
SPW = B // NW

NM = NF
NK = NM + 2


def _sc_counts(active_as):
    mesh = plsc.VectorSubcoreMesh(core_axis_name="c", subcore_axis_name="s")

    @functools.partial(
        pl.kernel,
        out_type=jax.ShapeDtypeStruct((B, AD), jnp.float32),
        mesh=mesh,
        compiler_params=pltpu.CompilerParams(needs_layout_passes=False,
                                             use_tc_tiling_on_sc=False),
        scratch_types=[
            pltpu.VMEM((SPW, NACT), jnp.int32),
            pltpu.VMEM((SPW, AD), jnp.float32),
        ],
    )
    def k(act_hbm, cnt_hbm, act_v, cnt_v):
        wid = lax.axis_index("s") * 2 + lax.axis_index("c")
        base = wid * SPW
        pltpu.sync_copy(act_hbm.at[pl.ds(base, SPW)], act_v)
        zeros16 = jnp.zeros((16,), jnp.float32)
        ones16 = jnp.ones((16,), jnp.float32)

        def zero_body(s, _):
            for v in range(AD // 16):
                cnt_v[s, pl.ds(v * 16, 16)] = zeros16
            return 0

        lax.fori_loop(0, SPW, zero_body, 0)

        def scat_body(s, _):
            svec = jnp.full((16,), s, jnp.int32)
            for v in range(NACT // 16):
                idx = act_v[s, pl.ds(v * 16, 16)]
                plsc.addupdate_scatter(cnt_v, [svec, idx], ones16)
            return 0

        lax.fori_loop(0, SPW, scat_body, 0)
        pltpu.sync_copy(cnt_v, cnt_hbm.at[pl.ds(base, SPW)])

    return k(active_as)


def _stats_body(c_ref, ft3, flb_ref, r1_ref, r2_ref, lb_ref, ftbuf, sems):
    q = pl.program_id(0)

    def ft_copy(qq, slot):
        return pltpu.make_async_copy(ft3.at[:, pl.ds(qq, 1), :],
                                     ftbuf.at[slot], sems.at[slot])

    @pl.when(q == 0)
    def _():
        for s in range(4):
            ft_copy(s, s).start()

    slot = lax.rem(q, 4)
    ft_copy(q, slot).wait()
    fb = ftbuf[slot].reshape(B, AD)
    cb = c_ref[...]
    t = cb * fb
    ones = jnp.ones((1, B), jnp.float32)
    dn = (((1,), (0,)), ((), ()))
    r1_ref[0] = lax.dot_general(ones, t, dn,
                                preferred_element_type=jnp.float32)
    r2_ref[0] = lax.dot_general(ones, t * fb, dn,
                                preferred_element_type=jnp.float32)

    @pl.when(q <= NF - 5)
    def _():
        ft_copy(q + 4, slot).start()

    @pl.when(q == NF - 1)
    def _():
        flb = flb_ref[...]
        lb_ref[0:1, :] = jnp.sum(flb, axis=0, keepdims=True)
        lb_ref[1:2, :] = jnp.sum(flb * flb, axis=0, keepdims=True)


def _stats_call(c2d, ft3, flb):
    return pl.pallas_call(
        _stats_body,
        grid=(NF,),
        in_specs=[
            pl.BlockSpec((B, AD), lambda q: (0, 0)),
            pl.BlockSpec(memory_space=pl.ANY),
            pl.BlockSpec((B, NLB), lambda q: (0, 0)),
        ],
        out_specs=[
            pl.BlockSpec((1, 1, AD), lambda q: (q, 0, 0)),
            pl.BlockSpec((1, 1, AD), lambda q: (q, 0, 0)),
            pl.BlockSpec((2, NLB), lambda q: (0, 0)),
        ],
        out_shape=[
            jax.ShapeDtypeStruct((NF, 1, AD), jnp.float32),
            jax.ShapeDtypeStruct((NF, 1, AD), jnp.float32),
            jax.ShapeDtypeStruct((2, NLB), jnp.float32),
        ],
        scratch_shapes=[
            pltpu.VMEM((4, B, 1, AD), jnp.float32),
            pltpu.SemaphoreType.DMA((4,)),
        ],
    )(c2d, ft3, flb)


def _mm_body(asm, bsm, ft3, m, flb, act, alb, blb, ahd, bhd, b1r, lnw, lnb,
             w4, b4r, w1r, out_ref, acc, wbuf, ftbuf, wlb, wact, sems,
             ftsems, semlb, semact):
    k = pl.program_id(0)
    dn = (((1,), (0,)), ((), ()))

    def wk_copy(g, slot):
        return pltpu.make_async_copy(
            w1r.at[pl.ds(2, AD), pl.ds(g, 1), :], wbuf.at[slot],
            sems.at[slot])

    def ft_copy(g, slot):
        return pltpu.make_async_copy(ft3.at[:, pl.ds(g, 1), :],
                                     ftbuf.at[slot], ftsems.at[slot])

    @pl.when(k == 0)
    def _():
        acc[...] = jnp.zeros((B, HID), jnp.float32)
        for s in range(4):
            wk_copy(s, s).start()
            ft_copy(s, s).start()
        pltpu.make_async_copy(w1r.at[pl.ds(0, 2), :, :], wlb, semlb).start()
        pltpu.make_async_copy(w1r.at[pl.ds(RW1 - 8, 8), :, :], wact,
                              semact).start()

    @pl.when(k <= NM - 1)
    def _():
        slot = lax.rem(k, 4)
        wk_copy(k, slot).wait()
        ft_copy(k, slot).wait()
        a = asm[k]
        b = bsm[k]
        x = m[...] * (ftbuf[slot].reshape(B, AD) * a + b)
        wv = wbuf[slot].reshape(AD, HID)
        acc[...] += lax.dot_general(x, wv, dn,
                                    preferred_element_type=jnp.float32)

        @pl.when(k <= NM - 5)
        def _():
            wk_copy(k + 4, slot).start()
            ft_copy(k + 4, slot).start()

    @pl.when(k == NM)
    def _():
        pltpu.make_async_copy(w1r.at[pl.ds(0, 2), :, :], wlb, semlb).wait()
        x0 = jnp.concatenate(
            [flb[:, NF:] * alb[...] + blb[...],
             flb[:, :NF] * ahd[...] + bhd[...]], axis=1)
        wv = wlb[...].reshape(NLB, HID)
        acc[...] += lax.dot_general(x0.astype(jnp.bfloat16),
                                    wv.astype(jnp.bfloat16), dn,
                                    preferred_element_type=jnp.float32)

    @pl.when(k == NK - 1)
    def _():
        pltpu.make_async_copy(w1r.at[pl.ds(RW1 - 8, 8), :, :], wact,
                              semact).wait()
        wv = wact[...].reshape(AD, HID)
        acc[...] += lax.dot_general(act[...].astype(jnp.bfloat16),
                                    wv.astype(jnp.bfloat16), dn,
                                    preferred_element_type=jnp.float32)
        h = acc[...] + b1r[...]
        h = jnp.where(h > 0, h, jnp.exp(jnp.minimum(h, 0.0)) - 1.0)
        mu = jnp.mean(h, axis=1, keepdims=True)
        hc = h - mu
        var = jnp.mean(hc * hc, axis=1, keepdims=True)
        hn = hc * lax.rsqrt(var + 1e-5) * lnw[...] + lnb[...]
        out_ref[...] = lax.dot_general(
            hn.astype(jnp.bfloat16), w4[...].astype(jnp.bfloat16), dn,
            preferred_element_type=jnp.float32) + b4r[...]


def _mm_call(alpha, beta, ft3, m, flb, act, alb, blb, ahd, bhd, b1r, lnw,
             lnb, W4, b4r, W1r):
    def full(shape):
        return pl.BlockSpec(shape, lambda k: tuple(0 for _ in shape))

    return pl.pallas_call(
        _mm_body,
        grid=(NK,),
        in_specs=[
            pl.BlockSpec(memory_space=pltpu.SMEM),
            pl.BlockSpec(memory_space=pltpu.SMEM),
            pl.BlockSpec(memory_space=pl.ANY),
            full((B, AD)),
            full((B, NLB)),
            full((B, AD)),
            full((1, NF)),
            full((1, NF)),
            full((1, NF)),
            full((1, NF)),
            full((1, HID)),
            full((1, HID)),
            full((1, HID)),
            full((HID, AD)),
            full((1, AD)),
            pl.BlockSpec(memory_space=pl.ANY),
        ],
        out_specs=pl.BlockSpec((B, AD), lambda k: (0, 0)),
        out_shape=jax.ShapeDtypeStruct((B, AD), jnp.float32),
        scratch_shapes=[
            pltpu.VMEM((B, HID), jnp.float32),
            pltpu.VMEM((4, AD, 1, HID), jnp.float32),
            pltpu.VMEM((4, B, 1, AD), jnp.float32),
            pltpu.VMEM((2, NF, HID), jnp.float32),
            pltpu.VMEM((8, NF, HID), jnp.float32),
            pltpu.SemaphoreType.DMA((4,)),
            pltpu.SemaphoreType.DMA((4,)),
            pltpu.SemaphoreType.DMA,
            pltpu.SemaphoreType.DMA,
        ],
    )(alpha, beta, ft3, m, flb, act, alb, blb, ahd, bhd, b1r, lnw, lnb,
      W4, b4r, W1r)


def kernel(feature_lb, feature_as, action, active_as, bn_as_w, bn_as_b,
           bn_lb_w, bn_lb_b, W1, b1, ln1_w, ln1_b, W4, b4):
    c2d = _sc_counts(active_as)

    ft3 = jnp.transpose(feature_as, (0, 2, 1))
    r1, r2, lbs = _stats_call(c2d, ft3, feature_lb)

    S1 = jnp.sum(r1.reshape(NF, AD), axis=1)
    S2 = jnp.sum(r2.reshape(NF, AD), axis=1)
    n_as = jnp.float32(B + B * NACT)
    mean_as = (S1 + lbs[0, :NF]) / n_as
    var_as = (S2 + lbs[1, :NF]) / n_as - mean_as * mean_as
    alpha_as = bn_as_w * lax.rsqrt(var_as + 1e-5)
    beta_as = bn_as_b - mean_as * alpha_as

    mean_lb = lbs[0, NF:] / B
    var_lb = lbs[1, NF:] / B - mean_lb * mean_lb
    alpha_lb = bn_lb_w * lax.rsqrt(var_lb + 1e-5)
    beta_lb = bn_lb_b - mean_lb * alpha_lb

    m = jnp.minimum(c2d, 1.0)
    W1r = W1.reshape(RW1, NF, HID)

    return _mm_call(alpha_as, beta_as, ft3, m, feature_lb, action,
                    alpha_lb[None, :], beta_lb[None, :], alpha_as[None, :],
                    beta_as[None, :], b1[None, :], ln1_w[None, :],
                    ln1_b[None, :], W4, b4[None, :], W1r)
